# Initial kernel scaffold; baseline (speedup 1.0000x reference)
import jax
import jax.numpy as jnp
from jax.experimental import pallas as pl


def kernel(kp, step):
    raise NotImplementedError("write your pallas kernel here")


# speedup vs baseline: 4.5759x; 4.5759x over previous
import functools

import jax
import jax.numpy as jnp
from jax import lax
from jax.experimental import pallas as pl
from jax.experimental.pallas import tpu as pltpu, tpu_sc as plsc

_B = 16384
_TABLE = 1024
_L = 16


def _make_gather():
    info = plsc.get_sparse_core_info()
    nc, ns = info.num_cores, info.num_subcores
    nw = nc * ns
    b_per_w = _B // nw
    mesh = plsc.VectorSubcoreMesh(core_axis_name="c", subcore_axis_name="s")

    @functools.partial(
        pl.kernel,
        mesh=mesh,
        compiler_params=pltpu.CompilerParams(needs_layout_passes=False),
        out_type=jax.ShapeDtypeStruct((_B,), jnp.float32),
        scratch_types=[
            pltpu.VMEM((_TABLE,), jnp.float32),
            pltpu.VMEM((b_per_w,), jnp.int32),
            pltpu.VMEM((b_per_w,), jnp.float32),
        ],
    )
    def gather_k(kp_hbm, step_hbm, out_hbm, table_v, idx_v, res_v):
        wid = lax.axis_index("s") * nc + lax.axis_index("c")
        base = wid * b_per_w
        pltpu.sync_copy(kp_hbm, table_v)
        pltpu.sync_copy(step_hbm.at[pl.ds(base, b_per_w)], idx_v)
        for i in range(b_per_w // _L):
            idx = idx_v[pl.ds(i * _L, _L)]
            res_v[pl.ds(i * _L, _L)] = plsc.load_gather(table_v, [idx])
        pltpu.sync_copy(res_v, out_hbm.at[pl.ds(base, b_per_w)])

    return gather_k


def kernel(kp, step):
    kp_pad = jnp.zeros((_TABLE,), jnp.float32).at[: kp.shape[0]].set(kp)
    return _make_gather()(kp_pad, step.astype(jnp.int32))


# speedup vs baseline: 4.6457x; 1.0152x over previous
import functools

import jax
import jax.numpy as jnp
from jax import lax
from jax.experimental import pallas as pl
from jax.experimental.pallas import tpu as pltpu, tpu_sc as plsc

_B = 16384
_TABLE = 1000
_TABLE_PAD = 1024
_L = 16


def _make_gather():
    info = plsc.get_sparse_core_info()
    nc, ns = info.num_cores, info.num_subcores
    nw = nc * ns
    b_per_w = _B // nw
    mesh = plsc.VectorSubcoreMesh(core_axis_name="c", subcore_axis_name="s")

    @functools.partial(
        pl.kernel,
        mesh=mesh,
        compiler_params=pltpu.CompilerParams(needs_layout_passes=False),
        out_type=jax.ShapeDtypeStruct((_B,), jnp.float32),
        scratch_types=[
            pltpu.VMEM((_TABLE_PAD,), jnp.float32),
            pltpu.VMEM((b_per_w,), jnp.int32),
            pltpu.VMEM((b_per_w,), jnp.float32),
            pltpu.SemaphoreType.DMA,
            pltpu.SemaphoreType.DMA,
        ],
    )
    def gather_k(kp_hbm, step_hbm, out_hbm, table_v, idx_v, res_v, sem_t, sem_i):
        wid = lax.axis_index("s") * nc + lax.axis_index("c")
        base = wid * b_per_w
        c_t = pltpu.async_copy(kp_hbm, table_v.at[pl.ds(0, _TABLE)], sem_t)
        c_i = pltpu.async_copy(step_hbm.at[pl.ds(base, b_per_w)], idx_v, sem_i)
        c_i.wait()
        c_t.wait()
        for i in range(b_per_w // _L):
            idx = idx_v[pl.ds(i * _L, _L)]
            res_v[pl.ds(i * _L, _L)] = plsc.load_gather(table_v, [idx])
        pltpu.sync_copy(res_v, out_hbm.at[pl.ds(base, b_per_w)])

    return gather_k


def kernel(kp, step):
    return _make_gather()(kp, step.astype(jnp.int32))


# speedup vs baseline: 4.6618x; 1.0035x over previous
import functools

import jax
import jax.numpy as jnp
from jax import lax
from jax.experimental import pallas as pl
from jax.experimental.pallas import tpu as pltpu, tpu_sc as plsc

_B = 16384
_TABLE = 1000
_TABLE_PAD = 1024
_L = 16


def _make_gather():
    info = plsc.get_sparse_core_info()
    nc, ns = info.num_cores, info.num_subcores
    nw = nc * ns
    b_per_w = _B // nw
    mesh = plsc.VectorSubcoreMesh(core_axis_name="c", subcore_axis_name="s")

    @functools.partial(
        pl.kernel,
        mesh=mesh,
        compiler_params=pltpu.CompilerParams(
            needs_layout_passes=False, skip_device_barrier=True
        ),
        out_type=jax.ShapeDtypeStruct((_B,), jnp.float32),
        scratch_types=[
            pltpu.VMEM((_TABLE_PAD,), jnp.float32),
            pltpu.VMEM((b_per_w,), jnp.int32),
            pltpu.VMEM((b_per_w,), jnp.float32),
            pltpu.SemaphoreType.DMA,
            pltpu.SemaphoreType.DMA,
        ],
    )
    def gather_k(kp_hbm, step_hbm, out_hbm, table_v, idx_v, res_v, sem_t, sem_i):
        wid = lax.axis_index("s") * nc + lax.axis_index("c")
        base = wid * b_per_w
        c_t = pltpu.async_copy(kp_hbm, table_v.at[pl.ds(0, _TABLE)], sem_t)
        c_i = pltpu.async_copy(step_hbm.at[pl.ds(base, b_per_w)], idx_v, sem_i)
        c_i.wait()
        c_t.wait()
        for i in range(b_per_w // _L):
            idx = idx_v[pl.ds(i * _L, _L)]
            res_v[pl.ds(i * _L, _L)] = plsc.load_gather(table_v, [idx])
        pltpu.sync_copy(res_v, out_hbm.at[pl.ds(base, b_per_w)])

    return gather_k


def kernel(kp, step):
    return _make_gather()(kp, step.astype(jnp.int32))


# speedup vs baseline: 4.6889x; 1.0058x over previous
import functools

import jax
import jax.numpy as jnp
from jax import lax
from jax.experimental import pallas as pl
from jax.experimental.pallas import tpu as pltpu, tpu_sc as plsc

_B = 16384
_TABLE = 1000
_TABLE_PAD = 1024
_L = 16


def _make_gather():
    info = plsc.get_sparse_core_info()
    nc, ns = info.num_cores, info.num_subcores
    nw = nc * ns
    b_per_w = _B // nw
    mesh = plsc.VectorSubcoreMesh(core_axis_name="c", subcore_axis_name="s")

    @functools.partial(
        pl.kernel,
        mesh=mesh,
        compiler_params=pltpu.CompilerParams(
            needs_layout_passes=False, skip_device_barrier=True
        ),
        out_type=jax.ShapeDtypeStruct((_B,), jnp.float32),
        scratch_types=[
            pltpu.VMEM((_TABLE_PAD,), jnp.float32),
            pltpu.VMEM((b_per_w,), jnp.int32),
            pltpu.VMEM((b_per_w,), jnp.float32),
            pltpu.SemaphoreType.DMA,
            pltpu.SemaphoreType.DMA,
        ],
    )
    def gather_k(kp_hbm, step_hbm, out_hbm, table_v, idx_v, res_v, sem_t, sem_i):
        wid = lax.axis_index("s") * nc + lax.axis_index("c")
        base = wid * b_per_w
        c_t = pltpu.async_copy(kp_hbm, table_v.at[pl.ds(0, _TABLE)], sem_t)
        c_i = pltpu.async_copy(step_hbm.at[pl.ds(base, b_per_w)], idx_v, sem_i)
        c_i.wait()
        c_t.wait()
        @pl.loop(0, b_per_w, step=_L, unroll=4)
        def _(off):
            idx = idx_v[pl.ds(off, _L)]
            res_v[pl.ds(off, _L)] = plsc.load_gather(table_v, [idx])
        pltpu.sync_copy(res_v, out_hbm.at[pl.ds(base, b_per_w)])

    return gather_k


def kernel(kp, step):
    return _make_gather()(kp, step.astype(jnp.int32))


# speedup vs baseline: 5.0889x; 1.0853x over previous
import functools

import jax
import jax.numpy as jnp
from jax import lax
from jax.experimental import pallas as pl
from jax.experimental.pallas import tpu as pltpu, tpu_sc as plsc

_B = 16384
_TABLE = 1000
_TABLE_PAD = 1024
_L = 16


def _make_gather():
    info = plsc.get_sparse_core_info()
    nc, ns = 1, info.num_subcores
    nw = nc * ns
    b_per_w = _B // nw
    mesh = plsc.VectorSubcoreMesh(
        core_axis_name="c", subcore_axis_name="s", num_cores=nc
    )

    @functools.partial(
        pl.kernel,
        mesh=mesh,
        compiler_params=pltpu.CompilerParams(
            needs_layout_passes=False, skip_device_barrier=True
        ),
        out_type=jax.ShapeDtypeStruct((_B,), jnp.float32),
        scratch_types=[
            pltpu.VMEM((_TABLE_PAD,), jnp.float32),
            pltpu.VMEM((b_per_w,), jnp.int32),
            pltpu.VMEM((b_per_w,), jnp.float32),
            pltpu.SemaphoreType.DMA,
            pltpu.SemaphoreType.DMA,
        ],
    )
    def gather_k(kp_hbm, step_hbm, out_hbm, table_v, idx_v, res_v, sem_t, sem_i):
        wid = lax.axis_index("s") * nc + lax.axis_index("c")
        base = wid * b_per_w
        c_t = pltpu.async_copy(kp_hbm, table_v.at[pl.ds(0, _TABLE)], sem_t)
        c_i = pltpu.async_copy(step_hbm.at[pl.ds(base, b_per_w)], idx_v, sem_i)
        c_i.wait()
        c_t.wait()
        @pl.loop(0, b_per_w, step=_L, unroll=4)
        def _(off):
            idx = idx_v[pl.ds(off, _L)]
            res_v[pl.ds(off, _L)] = plsc.load_gather(table_v, [idx])
        pltpu.sync_copy(res_v, out_hbm.at[pl.ds(base, b_per_w)])

    return gather_k


def kernel(kp, step):
    return _make_gather()(kp, step.astype(jnp.int32))
